# speedup vs baseline: 12.2208x; 12.2208x over previous
import functools

import jax
import jax.numpy as jnp
from jax.experimental import pallas as pl
from jax.experimental.pallas import tpu as pltpu

_VMEM_LIMIT = 64 * 1024 * 1024


def _gather_kernel(ids_ref, tbl_ref, o_ref, sem, *, rows):
    c = pl.program_id(0)
    base = c * rows

    def issue(mi, carry):
        pltpu.make_async_copy(
            tbl_ref.at[ids_ref[base + mi]], o_ref.at[mi], sem).start()
        return carry

    jax.lax.fori_loop(0, rows, issue, 0, unroll=True)
    pltpu.make_async_copy(
        tbl_ref.at[pl.ds(0, rows)], o_ref.at[pl.ds(0, rows)], sem).wait()


def _inproj_kernel(x_ref, w_ref, b_ref, o_ref, *, gsz):
    out = jnp.dot(x_ref[0], w_ref[...],
                  preferred_element_type=jnp.float32) + b_ref[...]
    o_ref[0, 0] = out[:, :gsz]
    o_ref[1, 0] = out[:, gsz:]


def _gru_kernel(xp_ref, wh_ref, bh_ref, len_ref, o_ref, h_ref, *, hsz):
    d = pl.program_id(0)
    t = pl.program_id(1)
    nt = pl.num_programs(1)

    @pl.when(t == 0)
    def _():
        h_ref[...] = jnp.zeros_like(h_ref)

    t_step = t + d * (nt - 1 - 2 * t)
    gx = xp_ref[0, 0]
    h_prev = h_ref[...]
    gh = jnp.dot(h_prev.astype(jnp.bfloat16), wh_ref[0],
                 preferred_element_type=jnp.float32) + bh_ref[0]
    r = jax.nn.sigmoid(gx[:, :hsz] + gh[:, :hsz])
    z = jax.nn.sigmoid(gx[:, hsz:2 * hsz] + gh[:, hsz:2 * hsz])
    n = jnp.tanh(gx[:, 2 * hsz:] + r * gh[:, 2 * hsz:])
    h_new = (1.0 - z) * n + z * h_prev
    valid = t_step < len_ref[...]
    o_ref[0] = jnp.where(valid, h_new, 0.0)
    h_ref[...] = jnp.where(valid, h_new, h_prev)


def _combine_kernel(y_ref, o_ref, *, eps):
    x = (y_ref[0] + y_ref[1]) * 0.5
    norm = jnp.sqrt(jnp.sum(x * x, axis=-1, keepdims=True)) + eps
    o_ref[...] = x * pl.reciprocal(norm, approx=False)


def kernel(embedding, ids, lengths,
           l0d0_w_ih, l0d0_b_ih, l0d0_w_hh, l0d0_b_hh,
           l0d1_w_ih, l0d1_b_ih, l0d1_w_hh, l0d1_b_hh):
    B, T = ids.shape
    V, D = embedding.shape
    G, H = l0d0_w_hh.shape
    M = T * B
    S = D // 128

    ids_tb = jnp.transpose(ids).reshape(M).astype(jnp.int32)
    tbl = embedding.astype(jnp.bfloat16).reshape(V, S, 128)

    x = pl.pallas_call(
        functools.partial(_gather_kernel, rows=B),
        out_shape=jax.ShapeDtypeStruct((M, S, 128), jnp.bfloat16),
        grid_spec=pltpu.PrefetchScalarGridSpec(
            num_scalar_prefetch=1,
            grid=(T,),
            in_specs=[pl.BlockSpec(memory_space=pl.ANY)],
            out_specs=pl.BlockSpec((B, S, 128), lambda c, ids: (c, 0, 0)),
            scratch_shapes=[pltpu.SemaphoreType.DMA]),
        compiler_params=pltpu.CompilerParams(
            dimension_semantics=("parallel",),
            vmem_limit_bytes=_VMEM_LIMIT),
    )(ids_tb, tbl)
    x = x.reshape(T, B, D)

    w_cat = jnp.concatenate(
        [l0d0_w_ih.T, l0d1_w_ih.T], axis=1).astype(jnp.bfloat16)
    b_cat = jnp.concatenate([l0d0_b_ih, l0d1_b_ih]).reshape(1, 2 * G)
    xp = pl.pallas_call(
        functools.partial(_inproj_kernel, gsz=G),
        out_shape=jax.ShapeDtypeStruct((2, T, B, G), jnp.float32),
        grid_spec=pltpu.PrefetchScalarGridSpec(
            num_scalar_prefetch=0,
            grid=(T,),
            in_specs=[pl.BlockSpec((1, B, D), lambda t: (t, 0, 0)),
                      pl.BlockSpec((D, 2 * G), lambda t: (0, 0)),
                      pl.BlockSpec((1, 2 * G), lambda t: (0, 0))],
            out_specs=pl.BlockSpec((2, 1, B, G), lambda t: (0, t, 0, 0))),
        compiler_params=pltpu.CompilerParams(
            dimension_semantics=("parallel",),
            vmem_limit_bytes=_VMEM_LIMIT),
    )(x, w_cat, b_cat)

    wh = jnp.stack([l0d0_w_hh.T, l0d1_w_hh.T]).astype(jnp.bfloat16)
    bh = jnp.stack([l0d0_b_hh, l0d1_b_hh]).reshape(2, 1, G)
    len2 = lengths.astype(jnp.int32).reshape(B, 1)
    t_eff = lambda d, t: t + d * (T - 1 - 2 * t)
    y = pl.pallas_call(
        functools.partial(_gru_kernel, hsz=H),
        out_shape=jax.ShapeDtypeStruct((2, B, T * H), jnp.float32),
        grid_spec=pltpu.PrefetchScalarGridSpec(
            num_scalar_prefetch=0,
            grid=(2, T),
            in_specs=[
                pl.BlockSpec((1, 1, B, G), lambda d, t: (d, t_eff(d, t), 0, 0)),
                pl.BlockSpec((1, H, G), lambda d, t: (d, 0, 0)),
                pl.BlockSpec((1, 1, G), lambda d, t: (d, 0, 0)),
                pl.BlockSpec((B, 1), lambda d, t: (0, 0))],
            out_specs=pl.BlockSpec((1, B, H), lambda d, t: (d, 0, t_eff(d, t))),
            scratch_shapes=[pltpu.VMEM((B, H), jnp.float32)]),
        compiler_params=pltpu.CompilerParams(
            dimension_semantics=("parallel", "arbitrary"),
            vmem_limit_bytes=_VMEM_LIMIT),
    )(xp, wh, bh, len2)

    tb = min(128, B)
    cap = pl.pallas_call(
        functools.partial(_combine_kernel, eps=1e-8),
        out_shape=jax.ShapeDtypeStruct((B, T * H), jnp.float32),
        grid_spec=pltpu.PrefetchScalarGridSpec(
            num_scalar_prefetch=0,
            grid=(B // tb, T),
            in_specs=[pl.BlockSpec((2, tb, H), lambda i, t: (0, i, t))],
            out_specs=pl.BlockSpec((tb, H), lambda i, t: (i, t))),
        compiler_params=pltpu.CompilerParams(
            dimension_semantics=("parallel", "arbitrary"),
            vmem_limit_bytes=_VMEM_LIMIT),
    )(y)

    return cap.reshape(B, T, H), lengths


# speedup vs baseline: 13.2804x; 1.0867x over previous
import functools

import jax
import jax.numpy as jnp
from jax.experimental import pallas as pl
from jax.experimental.pallas import tpu as pltpu

_VMEM_LIMIT = 64 * 1024 * 1024


def _gather_kernel(ids_ref, tbl_ref, o_ref, x_scr, sem, *, rows):
    c = pl.program_id(0)
    base = c * rows

    def issue(mi, carry):
        pltpu.make_async_copy(
            tbl_ref.at[ids_ref[base + mi]], x_scr.at[mi], sem).start()
        return carry

    jax.lax.fori_loop(0, rows, issue, 0, unroll=True)
    pltpu.make_async_copy(
        tbl_ref.at[pl.ds(0, rows)], x_scr.at[pl.ds(0, rows)], sem).wait()
    o_ref[...] = x_scr[...].astype(o_ref.dtype)


def _inproj_kernel(x_ref, w_ref, b_ref, o_ref, *, gsz):
    out = (jnp.dot(x_ref[0], w_ref[...],
                   preferred_element_type=jnp.float32)
           + b_ref[...]).astype(o_ref.dtype)
    o_ref[0, 0] = out[:, :gsz]
    o_ref[1, 0] = out[:, gsz:]


def _gru_kernel(xp_ref, wh_ref, bh_ref, len_ref, o_ref, h_ref, *, hsz):
    d = pl.program_id(0)
    t = pl.program_id(1)
    nt = pl.num_programs(1)

    @pl.when(t == 0)
    def _():
        h_ref[...] = jnp.zeros_like(h_ref)

    t_step = t + d * (nt - 1 - 2 * t)
    gx = xp_ref[0, 0].astype(jnp.float32)
    h_prev = h_ref[...]
    gh = jnp.dot(h_prev.astype(jnp.bfloat16), wh_ref[0],
                 preferred_element_type=jnp.float32) + bh_ref[0]
    r = jax.nn.sigmoid(gx[:, :hsz] + gh[:, :hsz])
    z = jax.nn.sigmoid(gx[:, hsz:2 * hsz] + gh[:, hsz:2 * hsz])
    n = jnp.tanh(gx[:, 2 * hsz:] + r * gh[:, 2 * hsz:])
    h_new = (1.0 - z) * n + z * h_prev
    valid = t_step < len_ref[...]
    o_ref[0] = jnp.where(valid, h_new, 0.0).astype(o_ref.dtype)
    h_ref[...] = jnp.where(valid, h_new, h_prev)


def _combine_kernel(y_ref, o_ref, *, eps):
    x = (y_ref[0].astype(jnp.float32) + y_ref[1].astype(jnp.float32)) * 0.5
    norm = jnp.sqrt(jnp.sum(x * x, axis=-1, keepdims=True)) + eps
    o_ref[...] = x * pl.reciprocal(norm, approx=False)


def kernel(embedding, ids, lengths,
           l0d0_w_ih, l0d0_b_ih, l0d0_w_hh, l0d0_b_hh,
           l0d1_w_ih, l0d1_b_ih, l0d1_w_hh, l0d1_b_hh):
    B, T = ids.shape
    V, D = embedding.shape
    G, H = l0d0_w_hh.shape
    M = T * B
    S = D // 128

    ids_tb = jnp.transpose(ids).reshape(M).astype(jnp.int32)
    tbl = embedding.reshape(V, S, 128)

    x = pl.pallas_call(
        functools.partial(_gather_kernel, rows=B),
        out_shape=jax.ShapeDtypeStruct((M, S, 128), jnp.bfloat16),
        grid_spec=pltpu.PrefetchScalarGridSpec(
            num_scalar_prefetch=1,
            grid=(T,),
            in_specs=[pl.BlockSpec(memory_space=pl.ANY)],
            out_specs=pl.BlockSpec((B, S, 128), lambda c, ids: (c, 0, 0)),
            scratch_shapes=[pltpu.VMEM((B, S, 128), jnp.float32),
                            pltpu.SemaphoreType.DMA]),
        compiler_params=pltpu.CompilerParams(
            dimension_semantics=("parallel",),
            vmem_limit_bytes=_VMEM_LIMIT),
    )(ids_tb, tbl)
    x = x.reshape(T, B, D)

    w_cat = jnp.concatenate(
        [l0d0_w_ih.T, l0d1_w_ih.T], axis=1).astype(jnp.bfloat16)
    b_cat = jnp.concatenate([l0d0_b_ih, l0d1_b_ih]).reshape(1, 2 * G)
    xp = pl.pallas_call(
        functools.partial(_inproj_kernel, gsz=G),
        out_shape=jax.ShapeDtypeStruct((2, T, B, G), jnp.bfloat16),
        grid_spec=pltpu.PrefetchScalarGridSpec(
            num_scalar_prefetch=0,
            grid=(T,),
            in_specs=[pl.BlockSpec((1, B, D), lambda t: (t, 0, 0)),
                      pl.BlockSpec((D, 2 * G), lambda t: (0, 0)),
                      pl.BlockSpec((1, 2 * G), lambda t: (0, 0))],
            out_specs=pl.BlockSpec((2, 1, B, G), lambda t: (0, t, 0, 0))),
        compiler_params=pltpu.CompilerParams(
            dimension_semantics=("parallel",),
            vmem_limit_bytes=_VMEM_LIMIT),
    )(x, w_cat, b_cat)

    wh = jnp.stack([l0d0_w_hh.T, l0d1_w_hh.T]).astype(jnp.bfloat16)
    bh = jnp.stack([l0d0_b_hh, l0d1_b_hh]).reshape(2, 1, G)
    len2 = lengths.astype(jnp.int32).reshape(B, 1)
    t_eff = lambda d, t: t + d * (T - 1 - 2 * t)
    y = pl.pallas_call(
        functools.partial(_gru_kernel, hsz=H),
        out_shape=jax.ShapeDtypeStruct((2, B, T * H), jnp.bfloat16),
        grid_spec=pltpu.PrefetchScalarGridSpec(
            num_scalar_prefetch=0,
            grid=(2, T),
            in_specs=[
                pl.BlockSpec((1, 1, B, G), lambda d, t: (d, t_eff(d, t), 0, 0)),
                pl.BlockSpec((1, H, G), lambda d, t: (d, 0, 0)),
                pl.BlockSpec((1, 1, G), lambda d, t: (d, 0, 0)),
                pl.BlockSpec((B, 1), lambda d, t: (0, 0))],
            out_specs=pl.BlockSpec((1, B, H), lambda d, t: (d, 0, t_eff(d, t))),
            scratch_shapes=[pltpu.VMEM((B, H), jnp.float32)]),
        compiler_params=pltpu.CompilerParams(
            dimension_semantics=("parallel", "arbitrary"),
            vmem_limit_bytes=_VMEM_LIMIT),
    )(xp, wh, bh, len2)

    tb = min(128, B)
    cap = pl.pallas_call(
        functools.partial(_combine_kernel, eps=1e-8),
        out_shape=jax.ShapeDtypeStruct((B, T * H), jnp.float32),
        grid_spec=pltpu.PrefetchScalarGridSpec(
            num_scalar_prefetch=0,
            grid=(B // tb, T),
            in_specs=[pl.BlockSpec((2, tb, H), lambda i, t: (0, i, t))],
            out_specs=pl.BlockSpec((tb, H), lambda i, t: (i, t))),
        compiler_params=pltpu.CompilerParams(
            dimension_semantics=("parallel", "arbitrary"),
            vmem_limit_bytes=_VMEM_LIMIT),
    )(y)

    return cap.reshape(B, T, H), lengths


# speedup vs baseline: 13.6592x; 1.0285x over previous
import functools

import jax
import jax.numpy as jnp
from jax.experimental import pallas as pl
from jax.experimental.pallas import tpu as pltpu

_VMEM_LIMIT = 64 * 1024 * 1024


def _gather_kernel(ids_ref, tbl_ref, o_ref, *, rows):
    c = pl.program_id(0)
    base = c * rows
    for mi in range(rows):
        o_ref[mi, 0] = tbl_ref[ids_ref[base + mi], 0]


def _inproj_kernel(x_ref, w_ref, b_ref, o_ref, *, gsz):
    out = (jnp.dot(x_ref[0].astype(jnp.bfloat16), w_ref[...],
                   preferred_element_type=jnp.float32)
           + b_ref[...]).astype(o_ref.dtype)
    o_ref[0, 0] = out[:, :gsz]
    o_ref[1, 0] = out[:, gsz:]


def _gru_kernel(xp_ref, wh_ref, bh_ref, len_ref, o_ref, h_ref, *, hsz):
    d = pl.program_id(0)
    t = pl.program_id(1)
    nt = pl.num_programs(1)

    @pl.when(t == 0)
    def _():
        h_ref[...] = jnp.zeros_like(h_ref)

    t_step = t + d * (nt - 1 - 2 * t)
    gx = xp_ref[0, 0].astype(jnp.float32)
    h_prev = h_ref[...]
    gh = jnp.dot(h_prev.astype(jnp.bfloat16), wh_ref[0],
                 preferred_element_type=jnp.float32) + bh_ref[0]
    r = jax.nn.sigmoid(gx[:, :hsz] + gh[:, :hsz])
    z = jax.nn.sigmoid(gx[:, hsz:2 * hsz] + gh[:, hsz:2 * hsz])
    n = jnp.tanh(gx[:, 2 * hsz:] + r * gh[:, 2 * hsz:])
    h_new = (1.0 - z) * n + z * h_prev
    valid = t_step < len_ref[...]
    o_ref[0] = jnp.where(valid, h_new, 0.0).astype(o_ref.dtype)
    h_ref[...] = jnp.where(valid, h_new, h_prev)


def _combine_kernel(y_ref, o_ref, *, eps):
    x = (y_ref[0].astype(jnp.float32) + y_ref[1].astype(jnp.float32)) * 0.5
    norm = jnp.sqrt(jnp.sum(x * x, axis=-1, keepdims=True)) + eps
    o_ref[...] = x * pl.reciprocal(norm, approx=False)


def kernel(embedding, ids, lengths,
           l0d0_w_ih, l0d0_b_ih, l0d0_w_hh, l0d0_b_hh,
           l0d1_w_ih, l0d1_b_ih, l0d1_w_hh, l0d1_b_hh):
    B, T = ids.shape
    V, D = embedding.shape
    G, H = l0d0_w_hh.shape
    M = T * B
    S = D // 128

    ids_tb = jnp.transpose(ids).reshape(M).astype(jnp.int32)
    tbl = embedding.reshape(V, 1, D)

    x = pl.pallas_call(
        functools.partial(_gather_kernel, rows=B),
        out_shape=jax.ShapeDtypeStruct((M, 1, D), jnp.float32),
        grid_spec=pltpu.PrefetchScalarGridSpec(
            num_scalar_prefetch=1,
            grid=(T,),
            in_specs=[pl.BlockSpec((V, 1, D), lambda c, ids: (0, 0, 0))],
            out_specs=pl.BlockSpec((B, 1, D), lambda c, ids: (c, 0, 0))),
        compiler_params=pltpu.CompilerParams(
            dimension_semantics=("parallel",),
            vmem_limit_bytes=_VMEM_LIMIT),
    )(ids_tb, tbl)
    x = x.reshape(T, B, D)

    w_cat = jnp.concatenate(
        [l0d0_w_ih.T, l0d1_w_ih.T], axis=1).astype(jnp.bfloat16)
    b_cat = jnp.concatenate([l0d0_b_ih, l0d1_b_ih]).reshape(1, 2 * G)
    xp = pl.pallas_call(
        functools.partial(_inproj_kernel, gsz=G),
        out_shape=jax.ShapeDtypeStruct((2, T, B, G), jnp.bfloat16),
        grid_spec=pltpu.PrefetchScalarGridSpec(
            num_scalar_prefetch=0,
            grid=(T,),
            in_specs=[pl.BlockSpec((1, B, D), lambda t: (t, 0, 0)),
                      pl.BlockSpec((D, 2 * G), lambda t: (0, 0)),
                      pl.BlockSpec((1, 2 * G), lambda t: (0, 0))],
            out_specs=pl.BlockSpec((2, 1, B, G), lambda t: (0, t, 0, 0))),
        compiler_params=pltpu.CompilerParams(
            dimension_semantics=("parallel",),
            vmem_limit_bytes=_VMEM_LIMIT),
    )(x, w_cat, b_cat)

    wh = jnp.stack([l0d0_w_hh.T, l0d1_w_hh.T]).astype(jnp.bfloat16)
    bh = jnp.stack([l0d0_b_hh, l0d1_b_hh]).reshape(2, 1, G)
    len2 = lengths.astype(jnp.int32).reshape(B, 1)
    t_eff = lambda d, t: t + d * (T - 1 - 2 * t)
    y = pl.pallas_call(
        functools.partial(_gru_kernel, hsz=H),
        out_shape=jax.ShapeDtypeStruct((2, B, T * H), jnp.bfloat16),
        grid_spec=pltpu.PrefetchScalarGridSpec(
            num_scalar_prefetch=0,
            grid=(2, T),
            in_specs=[
                pl.BlockSpec((1, 1, B, G), lambda d, t: (d, t_eff(d, t), 0, 0)),
                pl.BlockSpec((1, H, G), lambda d, t: (d, 0, 0)),
                pl.BlockSpec((1, 1, G), lambda d, t: (d, 0, 0)),
                pl.BlockSpec((B, 1), lambda d, t: (0, 0))],
            out_specs=pl.BlockSpec((1, B, H), lambda d, t: (d, 0, t_eff(d, t))),
            scratch_shapes=[pltpu.VMEM((B, H), jnp.float32)]),
        compiler_params=pltpu.CompilerParams(
            dimension_semantics=("parallel", "arbitrary"),
            vmem_limit_bytes=_VMEM_LIMIT),
    )(xp, wh, bh, len2)

    tb = min(128, B)
    cap = pl.pallas_call(
        functools.partial(_combine_kernel, eps=1e-8),
        out_shape=jax.ShapeDtypeStruct((B, T * H), jnp.float32),
        grid_spec=pltpu.PrefetchScalarGridSpec(
            num_scalar_prefetch=0,
            grid=(B // tb, T),
            in_specs=[pl.BlockSpec((2, tb, H), lambda i, t: (0, i, t))],
            out_specs=pl.BlockSpec((tb, H), lambda i, t: (i, t))),
        compiler_params=pltpu.CompilerParams(
            dimension_semantics=("parallel", "arbitrary"),
            vmem_limit_bytes=_VMEM_LIMIT),
    )(y)

    return cap.reshape(B, T, H), lengths


# speedup vs baseline: 15.2357x; 1.1154x over previous
import functools

import jax
import jax.numpy as jnp
from jax.experimental import pallas as pl
from jax.experimental.pallas import tpu as pltpu

_VMEM_LIMIT = 64 * 1024 * 1024


def _gather_kernel(ids_ref, tbl_ref, o_ref, *, rows, sc):
    c = pl.program_id(0)
    base = c * rows
    for mi in range(rows):
        i = pl.multiple_of(ids_ref[base + mi], sc)
        o_ref[pl.ds(sc * mi, sc), :] = tbl_ref[pl.ds(i, sc), :]


def _inproj_kernel(x_ref, w_ref, b_ref, o_ref, *, gsz):
    out = (jnp.dot(x_ref[0].astype(jnp.bfloat16), w_ref[...],
                   preferred_element_type=jnp.float32)
           + b_ref[...]).astype(o_ref.dtype)
    o_ref[0, 0] = out[:, :gsz]
    o_ref[1, 0] = out[:, gsz:]


def _gru_kernel(xp_ref, wh_ref, bh_ref, len_ref, o_ref, h_ref, *, hsz):
    d = pl.program_id(0)
    t = pl.program_id(1)
    nt = pl.num_programs(1)

    @pl.when(t == 0)
    def _():
        h_ref[...] = jnp.zeros_like(h_ref)

    t_step = t + d * (nt - 1 - 2 * t)
    gx = xp_ref[0, 0].astype(jnp.float32)
    h_prev = h_ref[...]
    gh = jnp.dot(h_prev.astype(jnp.bfloat16), wh_ref[0],
                 preferred_element_type=jnp.float32) + bh_ref[0]
    r = jax.nn.sigmoid(gx[:, :hsz] + gh[:, :hsz])
    z = jax.nn.sigmoid(gx[:, hsz:2 * hsz] + gh[:, hsz:2 * hsz])
    n = jnp.tanh(gx[:, 2 * hsz:] + r * gh[:, 2 * hsz:])
    h_new = (1.0 - z) * n + z * h_prev
    valid = t_step < len_ref[...]
    o_ref[0] = jnp.where(valid, h_new, 0.0).astype(o_ref.dtype)
    h_ref[...] = jnp.where(valid, h_new, h_prev)


def _combine_kernel(y_ref, o_ref, *, eps):
    x = (y_ref[0].astype(jnp.float32) + y_ref[1].astype(jnp.float32)) * 0.5
    norm = jnp.sqrt(jnp.sum(x * x, axis=-1, keepdims=True)) + eps
    o_ref[...] = x * pl.reciprocal(norm, approx=False)


def kernel(embedding, ids, lengths,
           l0d0_w_ih, l0d0_b_ih, l0d0_w_hh, l0d0_b_hh,
           l0d1_w_ih, l0d1_b_ih, l0d1_w_hh, l0d1_b_hh):
    B, T = ids.shape
    V, D = embedding.shape
    G, H = l0d0_w_hh.shape
    M = T * B
    S = D // 128

    ids_tb = (jnp.transpose(ids).reshape(M) * S).astype(jnp.int32)
    tbl = embedding.reshape(V * S, 128)

    x = pl.pallas_call(
        functools.partial(_gather_kernel, rows=B, sc=S),
        out_shape=jax.ShapeDtypeStruct((M * S, 128), jnp.float32),
        grid_spec=pltpu.PrefetchScalarGridSpec(
            num_scalar_prefetch=1,
            grid=(T,),
            in_specs=[pl.BlockSpec((V * S, 128), lambda c, ids: (0, 0))],
            out_specs=pl.BlockSpec((B * S, 128), lambda c, ids: (c, 0))),
        compiler_params=pltpu.CompilerParams(
            dimension_semantics=("parallel",),
            vmem_limit_bytes=_VMEM_LIMIT),
    )(ids_tb, tbl)
    x = x.reshape(T, B, D)

    w_cat = jnp.concatenate(
        [l0d0_w_ih.T, l0d1_w_ih.T], axis=1).astype(jnp.bfloat16)
    b_cat = jnp.concatenate([l0d0_b_ih, l0d1_b_ih]).reshape(1, 2 * G)
    xp = pl.pallas_call(
        functools.partial(_inproj_kernel, gsz=G),
        out_shape=jax.ShapeDtypeStruct((2, T, B, G), jnp.bfloat16),
        grid_spec=pltpu.PrefetchScalarGridSpec(
            num_scalar_prefetch=0,
            grid=(T,),
            in_specs=[pl.BlockSpec((1, B, D), lambda t: (t, 0, 0)),
                      pl.BlockSpec((D, 2 * G), lambda t: (0, 0)),
                      pl.BlockSpec((1, 2 * G), lambda t: (0, 0))],
            out_specs=pl.BlockSpec((2, 1, B, G), lambda t: (0, t, 0, 0))),
        compiler_params=pltpu.CompilerParams(
            dimension_semantics=("parallel",),
            vmem_limit_bytes=_VMEM_LIMIT),
    )(x, w_cat, b_cat)

    wh = jnp.stack([l0d0_w_hh.T, l0d1_w_hh.T]).astype(jnp.bfloat16)
    bh = jnp.stack([l0d0_b_hh, l0d1_b_hh]).reshape(2, 1, G)
    len2 = lengths.astype(jnp.int32).reshape(B, 1)
    t_eff = lambda d, t: t + d * (T - 1 - 2 * t)
    y = pl.pallas_call(
        functools.partial(_gru_kernel, hsz=H),
        out_shape=jax.ShapeDtypeStruct((2, B, T * H), jnp.bfloat16),
        grid_spec=pltpu.PrefetchScalarGridSpec(
            num_scalar_prefetch=0,
            grid=(2, T),
            in_specs=[
                pl.BlockSpec((1, 1, B, G), lambda d, t: (d, t_eff(d, t), 0, 0)),
                pl.BlockSpec((1, H, G), lambda d, t: (d, 0, 0)),
                pl.BlockSpec((1, 1, G), lambda d, t: (d, 0, 0)),
                pl.BlockSpec((B, 1), lambda d, t: (0, 0))],
            out_specs=pl.BlockSpec((1, B, H), lambda d, t: (d, 0, t_eff(d, t))),
            scratch_shapes=[pltpu.VMEM((B, H), jnp.float32)]),
        compiler_params=pltpu.CompilerParams(
            dimension_semantics=("parallel", "arbitrary"),
            vmem_limit_bytes=_VMEM_LIMIT),
    )(xp, wh, bh, len2)

    tb = min(128, B)
    cap = pl.pallas_call(
        functools.partial(_combine_kernel, eps=1e-8),
        out_shape=jax.ShapeDtypeStruct((B, T * H), jnp.float32),
        grid_spec=pltpu.PrefetchScalarGridSpec(
            num_scalar_prefetch=0,
            grid=(B // tb, T),
            in_specs=[pl.BlockSpec((2, tb, H), lambda i, t: (0, i, t))],
            out_specs=pl.BlockSpec((tb, H), lambda i, t: (i, t))),
        compiler_params=pltpu.CompilerParams(
            dimension_semantics=("parallel", "arbitrary"),
            vmem_limit_bytes=_VMEM_LIMIT),
    )(y)

    return cap.reshape(B, T, H), lengths


# speedup vs baseline: 21.0858x; 1.3840x over previous
import functools

import jax
import jax.numpy as jnp
from jax.experimental import pallas as pl
from jax.experimental.pallas import tpu as pltpu

_VMEM_LIMIT = 64 * 1024 * 1024


def _gather_kernel(ids_ref, tbl_ref, o_ref, *, rows, sc):
    c = pl.program_id(0)
    base = c * rows
    for mi in range(rows):
        i = pl.multiple_of(ids_ref[base + mi], sc)
        o_ref[pl.ds(sc * mi, sc), :] = tbl_ref[pl.ds(i, sc), :]


def _inproj_kernel(x_ref, w_ref, b_ref, o_ref, *, gsz, ts):
    nb, nd = x_ref.shape[1], x_ref.shape[2]
    xv = x_ref[...].reshape(ts * nb, nd).astype(jnp.bfloat16)
    out = (jnp.dot(xv, w_ref[...], preferred_element_type=jnp.float32)
           + b_ref[...]).astype(o_ref.dtype)
    o_ref[0] = out[:, :gsz].reshape(ts, nb, gsz)
    o_ref[1] = out[:, gsz:].reshape(ts, nb, gsz)


def _gru_kernel(xp_ref, wh_ref, bh_ref, len_ref, o_ref, h_ref, *, hsz, ts,
                steps):
    d = pl.program_id(0)
    tc = pl.program_id(1)

    @pl.when(tc == 0)
    def _():
        h_ref[...] = jnp.zeros_like(h_ref)

    wh = wh_ref[0]
    bh = bh_ref[0]
    lens = len_ref[...]
    h = h_ref[...]
    outs = []
    for j in range(ts):
        rj = j + d * (ts - 1 - 2 * j)
        sg = tc * ts + j
        t_step = sg + d * (steps - 1 - 2 * sg)
        gx = xp_ref[0, rj].astype(jnp.float32)
        gh = jnp.dot(h.astype(jnp.bfloat16), wh,
                     preferred_element_type=jnp.float32) + bh
        r = jax.nn.sigmoid(gx[:, :hsz] + gh[:, :hsz])
        z = jax.nn.sigmoid(gx[:, hsz:2 * hsz] + gh[:, hsz:2 * hsz])
        n = jnp.tanh(gx[:, 2 * hsz:] + r * gh[:, 2 * hsz:])
        h_new = (1.0 - z) * n + z * h
        valid = t_step < lens
        outs.append(jnp.where(valid, h_new, 0.0).astype(o_ref.dtype))
        h = jnp.where(valid, h_new, h)
    h_ref[...] = h

    @pl.when(d == 0)
    def _():
        for j in range(ts):
            o_ref[0, :, j, :] = outs[j]

    @pl.when(d == 1)
    def _():
        for j in range(ts):
            o_ref[0, :, ts - 1 - j, :] = outs[j]


def _combine_kernel(y_ref, o_ref, *, eps):
    x = (y_ref[0].astype(jnp.float32) + y_ref[1].astype(jnp.float32)) * 0.5
    norm = jnp.sqrt(jnp.sum(x * x, axis=-1, keepdims=True)) + eps
    o_ref[...] = x * pl.reciprocal(norm, approx=False)


def kernel(embedding, ids, lengths,
           l0d0_w_ih, l0d0_b_ih, l0d0_w_hh, l0d0_b_hh,
           l0d1_w_ih, l0d1_b_ih, l0d1_w_hh, l0d1_b_hh):
    B, T = ids.shape
    V, D = embedding.shape
    G, H = l0d0_w_hh.shape
    M = T * B
    S = D // 128

    ids_tb = (jnp.transpose(ids).reshape(M) * S).astype(jnp.int32)
    tbl = embedding.reshape(V * S, 128)

    x = pl.pallas_call(
        functools.partial(_gather_kernel, rows=B, sc=S),
        out_shape=jax.ShapeDtypeStruct((M * S, 128), jnp.float32),
        grid_spec=pltpu.PrefetchScalarGridSpec(
            num_scalar_prefetch=1,
            grid=(T,),
            in_specs=[pl.BlockSpec((V * S, 128), lambda c, ids: (0, 0))],
            out_specs=pl.BlockSpec((B * S, 128), lambda c, ids: (c, 0))),
        compiler_params=pltpu.CompilerParams(
            dimension_semantics=("parallel",),
            vmem_limit_bytes=_VMEM_LIMIT),
    )(ids_tb, tbl)
    x = x.reshape(T, B, D)

    w_cat = jnp.concatenate(
        [l0d0_w_ih.T, l0d1_w_ih.T], axis=1).astype(jnp.bfloat16)
    b_cat = jnp.concatenate([l0d0_b_ih, l0d1_b_ih]).reshape(1, 2 * G)
    ts_i = 4
    xp = pl.pallas_call(
        functools.partial(_inproj_kernel, gsz=G, ts=ts_i),
        out_shape=jax.ShapeDtypeStruct((2, T, B, G), jnp.bfloat16),
        grid_spec=pltpu.PrefetchScalarGridSpec(
            num_scalar_prefetch=0,
            grid=(T // ts_i,),
            in_specs=[pl.BlockSpec((ts_i, B, D), lambda t: (t, 0, 0)),
                      pl.BlockSpec((D, 2 * G), lambda t: (0, 0)),
                      pl.BlockSpec((1, 2 * G), lambda t: (0, 0))],
            out_specs=pl.BlockSpec((2, ts_i, B, G), lambda t: (0, t, 0, 0))),
        compiler_params=pltpu.CompilerParams(
            dimension_semantics=("parallel",),
            vmem_limit_bytes=_VMEM_LIMIT),
    )(x, w_cat, b_cat)

    wh = jnp.stack([l0d0_w_hh.T, l0d1_w_hh.T]).astype(jnp.bfloat16)
    bh = jnp.stack([l0d0_b_hh, l0d1_b_hh]).reshape(2, 1, G)
    len2 = lengths.astype(jnp.int32).reshape(B, 1)
    ts_g = 8
    ntc = T // ts_g
    tc_eff = lambda d, tc: tc + d * (ntc - 1 - 2 * tc)
    y = pl.pallas_call(
        functools.partial(_gru_kernel, hsz=H, ts=ts_g, steps=T),
        out_shape=jax.ShapeDtypeStruct((2, B, T, H), jnp.bfloat16),
        grid_spec=pltpu.PrefetchScalarGridSpec(
            num_scalar_prefetch=0,
            grid=(2, ntc),
            in_specs=[
                pl.BlockSpec((1, ts_g, B, G),
                             lambda d, tc: (d, tc_eff(d, tc), 0, 0)),
                pl.BlockSpec((1, H, G), lambda d, tc: (d, 0, 0)),
                pl.BlockSpec((1, 1, G), lambda d, tc: (d, 0, 0)),
                pl.BlockSpec((B, 1), lambda d, tc: (0, 0))],
            out_specs=pl.BlockSpec((1, B, ts_g, H),
                                   lambda d, tc: (d, 0, tc_eff(d, tc), 0)),
            scratch_shapes=[pltpu.VMEM((B, H), jnp.float32)]),
        compiler_params=pltpu.CompilerParams(
            dimension_semantics=("parallel", "arbitrary"),
            vmem_limit_bytes=_VMEM_LIMIT),
    )(xp, wh, bh, len2)

    tb = min(128, B)
    cap = pl.pallas_call(
        functools.partial(_combine_kernel, eps=1e-8),
        out_shape=jax.ShapeDtypeStruct((B, T, H), jnp.float32),
        grid_spec=pltpu.PrefetchScalarGridSpec(
            num_scalar_prefetch=0,
            grid=(B // tb, ntc),
            in_specs=[pl.BlockSpec((2, tb, ts_g, H),
                                   lambda i, tc: (0, i, tc, 0))],
            out_specs=pl.BlockSpec((tb, ts_g, H), lambda i, tc: (i, tc, 0))),
        compiler_params=pltpu.CompilerParams(
            dimension_semantics=("parallel", "arbitrary"),
            vmem_limit_bytes=_VMEM_LIMIT),
    )(y)

    return cap, lengths


# speedup vs baseline: 21.0977x; 1.0006x over previous
import functools

import jax
import jax.numpy as jnp
from jax.experimental import pallas as pl
from jax.experimental.pallas import tpu as pltpu

_VMEM_LIMIT = 64 * 1024 * 1024


def _gather_kernel(ids_ref, tbl_ref, o_ref, *, rows, sc):
    c = pl.program_id(0)
    base = c * rows
    for mi in range(rows):
        i = pl.multiple_of(ids_ref[base + mi], sc)
        o_ref[pl.ds(sc * mi, sc), :] = tbl_ref[pl.ds(i, sc), :]


def _inproj_kernel(x_ref, w_ref, b_ref, o_ref, *, gsz, ts):
    nb, nd = x_ref.shape[1], x_ref.shape[2]
    xv = x_ref[...].reshape(ts * nb, nd).astype(jnp.bfloat16)
    out = (jnp.dot(xv, w_ref[...], preferred_element_type=jnp.float32)
           + b_ref[...]).astype(o_ref.dtype)
    o_ref[0] = out[:, :gsz].reshape(ts, nb, gsz)
    o_ref[1] = out[:, gsz:].reshape(ts, nb, gsz)


def _gru_kernel(xp_ref, wh_ref, bh_ref, len_ref, o_ref, h_ref, *, hsz, ts,
                steps):
    d = pl.program_id(0)
    tc = pl.program_id(1)

    @pl.when(tc == 0)
    def _():
        h_ref[...] = jnp.zeros_like(h_ref)

    wh = wh_ref[0]
    bh = bh_ref[0]
    lens = len_ref[...]
    h = h_ref[...]
    outs = []
    for j in range(ts):
        rj = j + d * (ts - 1 - 2 * j)
        sg = tc * ts + j
        t_step = sg + d * (steps - 1 - 2 * sg)
        gx = xp_ref[0, rj].astype(jnp.float32)
        gh = jnp.dot(h.astype(jnp.bfloat16), wh,
                     preferred_element_type=jnp.float32) + bh
        r = jax.nn.sigmoid(gx[:, :hsz] + gh[:, :hsz])
        z = jax.nn.sigmoid(gx[:, hsz:2 * hsz] + gh[:, hsz:2 * hsz])
        n = jnp.tanh(gx[:, 2 * hsz:] + r * gh[:, 2 * hsz:])
        h_new = (1.0 - z) * n + z * h
        valid = t_step < lens
        outs.append(jnp.where(valid, h_new, 0.0).astype(o_ref.dtype))
        h = jnp.where(valid, h_new, h)
    h_ref[...] = h

    @pl.when(d == 0)
    def _():
        for j in range(ts):
            o_ref[0, :, j, :] = outs[j]

    @pl.when(d == 1)
    def _():
        for j in range(ts):
            o_ref[0, :, ts - 1 - j, :] = outs[j]


def _combine_kernel(y_ref, o_ref, *, eps):
    x = (y_ref[0].astype(jnp.float32) + y_ref[1].astype(jnp.float32)) * 0.5
    norm = jnp.sqrt(jnp.sum(x * x, axis=-1, keepdims=True)) + eps
    o_ref[...] = x * pl.reciprocal(norm, approx=False)


def kernel(embedding, ids, lengths,
           l0d0_w_ih, l0d0_b_ih, l0d0_w_hh, l0d0_b_hh,
           l0d1_w_ih, l0d1_b_ih, l0d1_w_hh, l0d1_b_hh):
    B, T = ids.shape
    V, D = embedding.shape
    G, H = l0d0_w_hh.shape
    M = T * B
    S = D // 128

    ids_tb = (jnp.transpose(ids).reshape(M) * S).astype(jnp.int32)
    tbl = embedding.reshape(V * S, 128)

    x = pl.pallas_call(
        functools.partial(_gather_kernel, rows=B, sc=S),
        out_shape=jax.ShapeDtypeStruct((M * S, 128), jnp.float32),
        grid_spec=pltpu.PrefetchScalarGridSpec(
            num_scalar_prefetch=1,
            grid=(T,),
            in_specs=[pl.BlockSpec((V * S, 128), lambda c, ids: (0, 0))],
            out_specs=pl.BlockSpec((B * S, 128), lambda c, ids: (c, 0))),
        compiler_params=pltpu.CompilerParams(
            dimension_semantics=("arbitrary",),
            vmem_limit_bytes=_VMEM_LIMIT),
    )(ids_tb, tbl)
    x = x.reshape(T, B, D)

    w_cat = jnp.concatenate(
        [l0d0_w_ih.T, l0d1_w_ih.T], axis=1).astype(jnp.bfloat16)
    b_cat = jnp.concatenate([l0d0_b_ih, l0d1_b_ih]).reshape(1, 2 * G)
    ts_i = 4
    xp = pl.pallas_call(
        functools.partial(_inproj_kernel, gsz=G, ts=ts_i),
        out_shape=jax.ShapeDtypeStruct((2, T, B, G), jnp.bfloat16),
        grid_spec=pltpu.PrefetchScalarGridSpec(
            num_scalar_prefetch=0,
            grid=(T // ts_i,),
            in_specs=[pl.BlockSpec((ts_i, B, D), lambda t: (t, 0, 0)),
                      pl.BlockSpec((D, 2 * G), lambda t: (0, 0)),
                      pl.BlockSpec((1, 2 * G), lambda t: (0, 0))],
            out_specs=pl.BlockSpec((2, ts_i, B, G), lambda t: (0, t, 0, 0))),
        compiler_params=pltpu.CompilerParams(
            dimension_semantics=("parallel",),
            vmem_limit_bytes=_VMEM_LIMIT),
    )(x, w_cat, b_cat)

    wh = jnp.stack([l0d0_w_hh.T, l0d1_w_hh.T]).astype(jnp.bfloat16)
    bh = jnp.stack([l0d0_b_hh, l0d1_b_hh]).reshape(2, 1, G)
    len2 = lengths.astype(jnp.int32).reshape(B, 1)
    ts_g = 8
    ntc = T // ts_g
    tc_eff = lambda d, tc: tc + d * (ntc - 1 - 2 * tc)
    y = pl.pallas_call(
        functools.partial(_gru_kernel, hsz=H, ts=ts_g, steps=T),
        out_shape=jax.ShapeDtypeStruct((2, B, T, H), jnp.bfloat16),
        grid_spec=pltpu.PrefetchScalarGridSpec(
            num_scalar_prefetch=0,
            grid=(2, ntc),
            in_specs=[
                pl.BlockSpec((1, ts_g, B, G),
                             lambda d, tc: (d, tc_eff(d, tc), 0, 0)),
                pl.BlockSpec((1, H, G), lambda d, tc: (d, 0, 0)),
                pl.BlockSpec((1, 1, G), lambda d, tc: (d, 0, 0)),
                pl.BlockSpec((B, 1), lambda d, tc: (0, 0))],
            out_specs=pl.BlockSpec((1, B, ts_g, H),
                                   lambda d, tc: (d, 0, tc_eff(d, tc), 0)),
            scratch_shapes=[pltpu.VMEM((B, H), jnp.float32)]),
        compiler_params=pltpu.CompilerParams(
            dimension_semantics=("parallel", "arbitrary"),
            vmem_limit_bytes=_VMEM_LIMIT),
    )(xp, wh, bh, len2)

    tb = min(128, B)
    cap = pl.pallas_call(
        functools.partial(_combine_kernel, eps=1e-8),
        out_shape=jax.ShapeDtypeStruct((B, T, H), jnp.float32),
        grid_spec=pltpu.PrefetchScalarGridSpec(
            num_scalar_prefetch=0,
            grid=(B // tb, ntc),
            in_specs=[pl.BlockSpec((2, tb, ts_g, H),
                                   lambda i, tc: (0, i, tc, 0))],
            out_specs=pl.BlockSpec((tb, ts_g, H), lambda i, tc: (i, tc, 0))),
        compiler_params=pltpu.CompilerParams(
            dimension_semantics=("parallel", "arbitrary"),
            vmem_limit_bytes=_VMEM_LIMIT),
    )(y)

    return cap, lengths


# speedup vs baseline: 21.2204x; 1.0058x over previous
import functools

import jax
import jax.numpy as jnp
from jax.experimental import pallas as pl
from jax.experimental.pallas import tpu as pltpu

_VMEM_LIMIT = 64 * 1024 * 1024


def _gather_kernel(ids_ref, tbl_ref, o_ref, scr, sems, *, rows, sc, nchunks):
    c = pl.program_id(0)
    rows_per = scr.shape[0] // nchunks

    @pl.when(c == 0)
    def _():
        for k in range(nchunks):
            pltpu.make_async_copy(
                tbl_ref.at[pl.ds(k * rows_per, rows_per)],
                scr.at[pl.ds(k * rows_per, rows_per)], sems.at[k]).start()
        for k in range(nchunks):
            pltpu.make_async_copy(
                tbl_ref.at[pl.ds(k * rows_per, rows_per)],
                scr.at[pl.ds(k * rows_per, rows_per)], sems.at[k]).wait()

    base = c * rows
    for mi in range(rows):
        i = pl.multiple_of(ids_ref[base + mi], sc)
        o_ref[pl.ds(sc * mi, sc), :] = scr[pl.ds(i, sc), :]


def _inproj_kernel(x_ref, w_ref, b_ref, o_ref, *, gsz, ts):
    nb, nd = x_ref.shape[1], x_ref.shape[2]
    xv = x_ref[...].reshape(ts * nb, nd).astype(jnp.bfloat16)
    out = (jnp.dot(xv, w_ref[...], preferred_element_type=jnp.float32)
           + b_ref[...]).astype(o_ref.dtype)
    o_ref[0] = out[:, :gsz].reshape(ts, nb, gsz)
    o_ref[1] = out[:, gsz:].reshape(ts, nb, gsz)


def _gru_kernel(xp_ref, wh_ref, bh_ref, len_ref, o_ref, h_ref, *, hsz, ts,
                steps):
    d = pl.program_id(0)
    tc = pl.program_id(1)

    @pl.when(tc == 0)
    def _():
        h_ref[...] = jnp.zeros_like(h_ref)

    wh = wh_ref[0]
    bh = bh_ref[0]
    lens = len_ref[...]
    h = h_ref[...]
    outs = []
    for j in range(ts):
        rj = j + d * (ts - 1 - 2 * j)
        sg = tc * ts + j
        t_step = sg + d * (steps - 1 - 2 * sg)
        gx = xp_ref[0, rj].astype(jnp.float32)
        gh = jnp.dot(h.astype(jnp.bfloat16), wh,
                     preferred_element_type=jnp.float32) + bh
        r = jax.nn.sigmoid(gx[:, :hsz] + gh[:, :hsz])
        z = jax.nn.sigmoid(gx[:, hsz:2 * hsz] + gh[:, hsz:2 * hsz])
        n = jnp.tanh(gx[:, 2 * hsz:] + r * gh[:, 2 * hsz:])
        h_new = (1.0 - z) * n + z * h
        valid = t_step < lens
        outs.append(jnp.where(valid, h_new, 0.0).astype(o_ref.dtype))
        h = jnp.where(valid, h_new, h)
    h_ref[...] = h

    @pl.when(d == 0)
    def _():
        for j in range(ts):
            o_ref[0, :, j, :] = outs[j]

    @pl.when(d == 1)
    def _():
        for j in range(ts):
            o_ref[0, :, ts - 1 - j, :] = outs[j]


def _combine_kernel(y_ref, o_ref, *, eps):
    x = (y_ref[0].astype(jnp.float32) + y_ref[1].astype(jnp.float32)) * 0.5
    norm = jnp.sqrt(jnp.sum(x * x, axis=-1, keepdims=True)) + eps
    o_ref[...] = x * pl.reciprocal(norm, approx=False)


def kernel(embedding, ids, lengths,
           l0d0_w_ih, l0d0_b_ih, l0d0_w_hh, l0d0_b_hh,
           l0d1_w_ih, l0d1_b_ih, l0d1_w_hh, l0d1_b_hh):
    B, T = ids.shape
    V, D = embedding.shape
    G, H = l0d0_w_hh.shape
    M = T * B
    S = D // 128

    ids_tb = (jnp.transpose(ids).reshape(M) * S).astype(jnp.int32)
    tbl = embedding.reshape(V * S, 128)

    x = pl.pallas_call(
        functools.partial(_gather_kernel, rows=B, sc=S, nchunks=8),
        out_shape=jax.ShapeDtypeStruct((M * S, 128), jnp.float32),
        grid_spec=pltpu.PrefetchScalarGridSpec(
            num_scalar_prefetch=1,
            grid=(T,),
            in_specs=[pl.BlockSpec(memory_space=pl.ANY)],
            out_specs=pl.BlockSpec((B * S, 128), lambda c, ids: (c, 0)),
            scratch_shapes=[pltpu.VMEM((V * S, 128), jnp.float32),
                            pltpu.SemaphoreType.DMA((8,))]),
        compiler_params=pltpu.CompilerParams(
            dimension_semantics=("arbitrary",),
            vmem_limit_bytes=_VMEM_LIMIT),
    )(ids_tb, tbl)
    x = x.reshape(T, B, D)

    w_cat = jnp.concatenate(
        [l0d0_w_ih.T, l0d1_w_ih.T], axis=1).astype(jnp.bfloat16)
    b_cat = jnp.concatenate([l0d0_b_ih, l0d1_b_ih]).reshape(1, 2 * G)
    ts_i = 4
    xp = pl.pallas_call(
        functools.partial(_inproj_kernel, gsz=G, ts=ts_i),
        out_shape=jax.ShapeDtypeStruct((2, T, B, G), jnp.bfloat16),
        grid_spec=pltpu.PrefetchScalarGridSpec(
            num_scalar_prefetch=0,
            grid=(T // ts_i,),
            in_specs=[pl.BlockSpec((ts_i, B, D), lambda t: (t, 0, 0)),
                      pl.BlockSpec((D, 2 * G), lambda t: (0, 0)),
                      pl.BlockSpec((1, 2 * G), lambda t: (0, 0))],
            out_specs=pl.BlockSpec((2, ts_i, B, G), lambda t: (0, t, 0, 0))),
        compiler_params=pltpu.CompilerParams(
            dimension_semantics=("parallel",),
            vmem_limit_bytes=_VMEM_LIMIT),
    )(x, w_cat, b_cat)

    wh = jnp.stack([l0d0_w_hh.T, l0d1_w_hh.T]).astype(jnp.bfloat16)
    bh = jnp.stack([l0d0_b_hh, l0d1_b_hh]).reshape(2, 1, G)
    len2 = lengths.astype(jnp.int32).reshape(B, 1)
    ts_g = 8
    ntc = T // ts_g
    tc_eff = lambda d, tc: tc + d * (ntc - 1 - 2 * tc)
    y = pl.pallas_call(
        functools.partial(_gru_kernel, hsz=H, ts=ts_g, steps=T),
        out_shape=jax.ShapeDtypeStruct((2, B, T, H), jnp.bfloat16),
        grid_spec=pltpu.PrefetchScalarGridSpec(
            num_scalar_prefetch=0,
            grid=(2, ntc),
            in_specs=[
                pl.BlockSpec((1, ts_g, B, G),
                             lambda d, tc: (d, tc_eff(d, tc), 0, 0)),
                pl.BlockSpec((1, H, G), lambda d, tc: (d, 0, 0)),
                pl.BlockSpec((1, 1, G), lambda d, tc: (d, 0, 0)),
                pl.BlockSpec((B, 1), lambda d, tc: (0, 0))],
            out_specs=pl.BlockSpec((1, B, ts_g, H),
                                   lambda d, tc: (d, 0, tc_eff(d, tc), 0)),
            scratch_shapes=[pltpu.VMEM((B, H), jnp.float32)]),
        compiler_params=pltpu.CompilerParams(
            dimension_semantics=("parallel", "arbitrary"),
            vmem_limit_bytes=_VMEM_LIMIT),
    )(xp, wh, bh, len2)

    tb = min(128, B)
    cap = pl.pallas_call(
        functools.partial(_combine_kernel, eps=1e-8),
        out_shape=jax.ShapeDtypeStruct((B, T, H), jnp.float32),
        grid_spec=pltpu.PrefetchScalarGridSpec(
            num_scalar_prefetch=0,
            grid=(B // tb, ntc),
            in_specs=[pl.BlockSpec((2, tb, ts_g, H),
                                   lambda i, tc: (0, i, tc, 0))],
            out_specs=pl.BlockSpec((tb, ts_g, H), lambda i, tc: (i, tc, 0))),
        compiler_params=pltpu.CompilerParams(
            dimension_semantics=("parallel", "arbitrary"),
            vmem_limit_bytes=_VMEM_LIMIT),
    )(y)

    return cap, lengths
